# Initial kernel scaffold; baseline (speedup 1.0000x reference)
import jax
import jax.numpy as jnp
from jax.experimental import pallas as pl


def kernel(nodes, neighbors, W, a_src, a_tgt, bias):
    raise NotImplementedError("write your pallas kernel here")


# speedup vs baseline: 1.6267x; 1.6267x over previous
import functools

import jax
import jax.numpy as jnp
from jax.experimental import pallas as pl


def _gat_block(nodes_ref, nbr_ref, w_ref, a_src_ref, a_tgt_ref, bias_ref,
               out_ref):
    w = w_ref[...]
    a_src = a_src_ref[0]
    a_tgt = a_tgt_ref[0]
    wa_src = jnp.dot(a_src, w, preferred_element_type=jnp.float32)
    wa_tgt = jnp.dot(a_tgt, w, preferred_element_type=jnp.float32)

    nodes = nodes_ref[...]
    nbr = nbr_ref[...]

    s_t = jnp.sum(nodes * wa_tgt, axis=-1)
    s_s = jnp.sum(nbr * wa_src[None], axis=-1)

    scores = s_s + s_t[:, None]
    scores = jnp.where(scores >= 0.0, scores, 0.2 * scores)
    m = jnp.max(scores, axis=1, keepdims=True)
    e = jnp.exp(scores - m)
    att = e / jnp.sum(e, axis=1, keepdims=True)

    agg = jnp.sum(nbr * att[..., None], axis=1)
    out = jnp.dot(agg, w.T, preferred_element_type=jnp.float32)
    out = out + bias_ref[...]
    out_ref[...] = jnp.where(out > 0.0, out, jnp.exp(out) - 1.0)


@functools.partial(jax.jit, static_argnames=())
def kernel(nodes, neighbors, W, a_src, a_tgt, bias):
    n, f_in = nodes.shape
    deg = neighbors.shape[1]
    f_out = W.shape[0]
    bn = 400
    grid = (n // bn,)
    bias2 = bias.reshape(1, f_out)
    return pl.pallas_call(
        _gat_block,
        grid=grid,
        in_specs=[
            pl.BlockSpec((bn, f_in), lambda i: (i, 0)),
            pl.BlockSpec((bn, deg, f_in), lambda i: (i, 0, 0)),
            pl.BlockSpec((f_out, f_in), lambda i: (0, 0)),
            pl.BlockSpec((1, 1, f_out), lambda i: (0, 0, 0)),
            pl.BlockSpec((1, 1, f_out), lambda i: (0, 0, 0)),
            pl.BlockSpec((1, f_out), lambda i: (0, 0)),
        ],
        out_specs=pl.BlockSpec((bn, f_out), lambda i: (i, 0)),
        out_shape=jax.ShapeDtypeStruct((n, f_out), jnp.float32),
    )(nodes, neighbors, W, a_src, a_tgt, bias2)


# speedup vs baseline: 2.0791x; 1.2781x over previous
import functools

import jax
import jax.numpy as jnp
from jax.experimental import pallas as pl


def _gat_block(nodes_ref, nbr_ref, w_ref, a_src_ref, a_tgt_ref, bias_ref,
               out_ref):
    w = w_ref[...]
    a_src = a_src_ref[0]
    a_tgt = a_tgt_ref[0]
    wa_src = jnp.dot(a_src, w, preferred_element_type=jnp.float32)
    wa_tgt = jnp.dot(a_tgt, w, preferred_element_type=jnp.float32)

    nodes = nodes_ref[...]
    nbr = nbr_ref[...]

    s_t = jnp.sum(nodes * wa_tgt, axis=-1)
    s_s = jnp.sum(nbr * wa_src[None], axis=-1)

    scores = s_s + s_t[:, None]
    scores = jnp.where(scores >= 0.0, scores, 0.2 * scores)
    e = jnp.exp(scores)
    att = e / jnp.sum(e, axis=1, keepdims=True)

    agg = jnp.sum(nbr * att[..., None], axis=1)
    out = jnp.dot(agg, w.T, preferred_element_type=jnp.float32)
    out = out + bias_ref[...]
    out_ref[...] = jnp.where(out > 0.0, out, jnp.exp(out) - 1.0)


@functools.partial(jax.jit, static_argnames=())
def kernel(nodes, neighbors, W, a_src, a_tgt, bias):
    n, f_in = nodes.shape
    deg = neighbors.shape[1]
    f_out = W.shape[0]
    bn = 400
    grid = (n // bn,)
    bias2 = bias.reshape(1, f_out)
    return pl.pallas_call(
        _gat_block,
        grid=grid,
        in_specs=[
            pl.BlockSpec((bn, f_in), lambda i: (i, 0)),
            pl.BlockSpec((bn, deg, f_in), lambda i: (i, 0, 0)),
            pl.BlockSpec((f_out, f_in), lambda i: (0, 0)),
            pl.BlockSpec((1, 1, f_out), lambda i: (0, 0, 0)),
            pl.BlockSpec((1, 1, f_out), lambda i: (0, 0, 0)),
            pl.BlockSpec((1, f_out), lambda i: (0, 0)),
        ],
        out_specs=pl.BlockSpec((bn, f_out), lambda i: (i, 0)),
        out_shape=jax.ShapeDtypeStruct((n, f_out), jnp.float32),
    )(nodes, neighbors, W, a_src, a_tgt, bias2)


# speedup vs baseline: 2.5479x; 1.2255x over previous
import functools

import jax
import jax.numpy as jnp
from jax.experimental import pallas as pl


def _gat_block(nodes_ref, nbr_ref, w_ref, a_src_ref, a_tgt_ref, bias_ref,
               out_ref):
    w = w_ref[...]
    a_src = a_src_ref[0]
    a_tgt = a_tgt_ref[0]
    wa_src = jnp.dot(a_src, w, preferred_element_type=jnp.float32)
    wa_tgt = jnp.dot(a_tgt, w, preferred_element_type=jnp.float32)

    nodes = nodes_ref[...]
    nbr = nbr_ref[...]

    s_t = jnp.sum(nodes * wa_tgt, axis=-1)
    s_s = jnp.sum(nbr * wa_src[None], axis=-1)

    scores = s_s + s_t[:, None]
    scores = jnp.where(scores >= 0.0, scores, 0.2 * scores)
    e = jnp.exp(scores)
    denom = jnp.sum(e, axis=1)

    num = jnp.sum(nbr * e[..., None], axis=1)
    out = jnp.dot(num, w.T, preferred_element_type=jnp.float32)
    out = out / (denom[:, None] + 1e-16) + bias_ref[...]
    out_ref[...] = jnp.where(out > 0.0, out, jnp.exp(out) - 1.0)


@functools.partial(jax.jit, static_argnames=())
def kernel(nodes, neighbors, W, a_src, a_tgt, bias):
    n, f_in = nodes.shape
    deg = neighbors.shape[1]
    f_out = W.shape[0]
    bn = 400
    grid = (n // bn,)
    bias2 = bias.reshape(1, f_out)
    return pl.pallas_call(
        _gat_block,
        grid=grid,
        in_specs=[
            pl.BlockSpec((bn, f_in), lambda i: (i, 0)),
            pl.BlockSpec((bn, deg, f_in), lambda i: (i, 0, 0)),
            pl.BlockSpec((f_out, f_in), lambda i: (0, 0)),
            pl.BlockSpec((1, 1, f_out), lambda i: (0, 0, 0)),
            pl.BlockSpec((1, 1, f_out), lambda i: (0, 0, 0)),
            pl.BlockSpec((1, f_out), lambda i: (0, 0)),
        ],
        out_specs=pl.BlockSpec((bn, f_out), lambda i: (i, 0)),
        out_shape=jax.ShapeDtypeStruct((n, f_out), jnp.float32),
    )(nodes, neighbors, W, a_src, a_tgt, bias2)


# speedup vs baseline: 2.7913x; 1.0955x over previous
import functools

import jax
import jax.numpy as jnp
from jax.experimental import pallas as pl


def _gat_block(nodes_ref, nbr_ref, w_ref, a_src_ref, a_tgt_ref, bias_ref,
               out_ref):
    w = w_ref[...]
    a_src = a_src_ref[0]
    a_tgt = a_tgt_ref[0]
    wa_src = jnp.dot(a_src, w, preferred_element_type=jnp.float32)
    wa_tgt = jnp.dot(a_tgt, w, preferred_element_type=jnp.float32)

    nodes = nodes_ref[...]
    nbr = nbr_ref[...]

    s_t = jnp.sum(nodes * wa_tgt, axis=-1)
    s_s = jnp.sum(nbr * wa_src[None], axis=-1)

    scores = s_s + s_t[:, None]
    scores = jnp.where(scores >= 0.0, scores, 0.2 * scores)
    e = jnp.exp(scores)
    denom = jnp.sum(e, axis=1)

    num = jnp.sum(nbr * e[..., None], axis=1)
    out = jnp.dot(num, w.T, preferred_element_type=jnp.float32)
    out = out / (denom[:, None] + 1e-16) + bias_ref[...]
    out_ref[...] = jnp.where(out > 0.0, out, jnp.exp(out) - 1.0)


@functools.partial(jax.jit, static_argnames=())
def kernel(nodes, neighbors, W, a_src, a_tgt, bias):
    n, f_in = nodes.shape
    deg = neighbors.shape[1]
    f_out = W.shape[0]
    bn = 1000
    grid = (n // bn,)
    bias2 = bias.reshape(1, f_out)
    return pl.pallas_call(
        _gat_block,
        grid=grid,
        in_specs=[
            pl.BlockSpec((bn, f_in), lambda i: (i, 0)),
            pl.BlockSpec((bn, deg, f_in), lambda i: (i, 0, 0)),
            pl.BlockSpec((f_out, f_in), lambda i: (0, 0)),
            pl.BlockSpec((1, 1, f_out), lambda i: (0, 0, 0)),
            pl.BlockSpec((1, 1, f_out), lambda i: (0, 0, 0)),
            pl.BlockSpec((1, f_out), lambda i: (0, 0)),
        ],
        out_specs=pl.BlockSpec((bn, f_out), lambda i: (i, 0)),
        out_shape=jax.ShapeDtypeStruct((n, f_out), jnp.float32),
    )(nodes, neighbors, W, a_src, a_tgt, bias2)
